# Initial kernel scaffold; baseline (speedup 1.0000x reference)
import jax
import jax.numpy as jnp
from jax.experimental import pallas as pl


def kernel(x, gate_w, gate_b, w1, b1, w2, b2):
    raise NotImplementedError("write your pallas kernel here")


# speedup vs baseline: 2.2418x; 2.2418x over previous
import functools
import jax
import jax.numpy as jnp
from jax import lax
from jax.experimental import pallas as pl
from jax.experimental.pallas import tpu as pltpu
from jax.experimental.pallas import tpu_sc as plsc

S, D, H, E, K = 2048, 1024, 4096, 8, 2
T = 256
NP = S * K + E * T
NT = NP // T
HB = 512
JH = H // HB

_NEG = -1e30
_INV_SQRT2 = 0.7071067811865476


def _gelu(a):
    return a * 0.5 * (1.0 + jax.lax.erf(a * _INV_SQRT2))


def _gate_body(x_ref, gw_ref, gb_ref, e0_ref, e1_ref, g0_ref, g1_ref):
    gt = jax.lax.dot_general(
        gw_ref[...], x_ref[...],
        dimension_numbers=(((0,), (1,)), ((), ())),
        preferred_element_type=jnp.float32) + gb_ref[...]
    s0 = jnp.full((1, S), _NEG, jnp.float32)
    e0 = jnp.zeros((1, S), jnp.int32)
    for e in range(E):
        g = gt[e:e + 1, :]
        upd = g > s0
        e0 = jnp.where(upd, e, e0)
        s0 = jnp.where(upd, g, s0)
    s1 = jnp.full((1, S), _NEG, jnp.float32)
    e1 = jnp.zeros((1, S), jnp.int32)
    for e in range(E):
        g = gt[e:e + 1, :]
        upd = (g > s1) & (e0 != e)
        e1 = jnp.where(upd, e, e1)
        s1 = jnp.where(upd, g, s1)
    z = jnp.exp(s1 - s0)
    g1 = z / (1.0 + z)
    e0_ref[...] = e0
    e1_ref[...] = e1
    g0_ref[...] = 1.0 - g1
    g1_ref[...] = g1


def _gate(x_flat, gate_w, gate_b):
    e0, e1, g0, g1 = pl.pallas_call(
        _gate_body,
        out_shape=[
            jax.ShapeDtypeStruct((1, S), jnp.int32),
            jax.ShapeDtypeStruct((1, S), jnp.int32),
            jax.ShapeDtypeStruct((1, S), jnp.float32),
            jax.ShapeDtypeStruct((1, S), jnp.float32),
        ],
    )(x_flat, gate_w, gate_b.reshape(E, 1))
    return e0.reshape(S), e1.reshape(S), g0.reshape(S), g1.reshape(S)


def _route_jax(e0, e1):
    eids = jnp.arange(E, dtype=jnp.int32)
    match = ((e0[:, None] == eids) | (e1[:, None] == eids)).astype(jnp.int32)
    counts = match.sum(0)
    pt = (counts + T - 1) // T
    tile_excl = jnp.concatenate(
        [jnp.zeros(1, jnp.int32), jnp.cumsum(pt)[:-1].astype(jnp.int32)])
    start = tile_excl * T
    rank = jnp.cumsum(match, axis=0) - match
    pos_te = start[None, :] + rank
    tok = jnp.arange(S, dtype=jnp.int32)
    pos0 = pos_te[tok, e0]
    pos1 = pos_te[tok, e1]
    disp_tok = jnp.zeros(NP, jnp.int32).at[pos0].set(tok).at[pos1].set(tok)
    te = jnp.clip(
        (jnp.arange(NT)[:, None] >= tile_excl[None, :]).astype(jnp.int32).sum(1) - 1,
        0, E - 1).astype(jnp.int32)
    return disp_tok, pos0, pos1, te


def _ffn1_body(te_ref, xg_ref, w1_ref, b1_ref, h_ref):
    del te_ref
    i = pl.program_id(1)
    xb = xg_ref[pl.ds(i * T, T), :]
    a = jnp.dot(xb, w1_ref[0], preferred_element_type=jnp.float32) + b1_ref[0]
    h_ref[...] = _gelu(a)


def _ffn1(te, xg, w1, b1):
    grid_spec = pltpu.PrefetchScalarGridSpec(
        num_scalar_prefetch=1,
        grid=(JH, NT),
        in_specs=[
            pl.BlockSpec((NP, D), lambda j, i, te: (0, 0)),
            pl.BlockSpec((1, D, HB), lambda j, i, te: (te[i], 0, j)),
            pl.BlockSpec((1, 1, HB), lambda j, i, te: (te[i], 0, j)),
        ],
        out_specs=pl.BlockSpec((T, HB), lambda j, i, te: (i, j)),
    )
    return pl.pallas_call(
        _ffn1_body,
        grid_spec=grid_spec,
        out_shape=jax.ShapeDtypeStruct((NP, H), jnp.float32),
    )(te, xg, w1, b1.reshape(E, 1, H))


def _ffn2_body(te_ref, h_ref, w2_ref, b2_ref, y_ref):
    del te_ref
    y_ref[...] = jnp.dot(
        h_ref[...], w2_ref[0], preferred_element_type=jnp.float32) + b2_ref[0]


def _ffn2(te, h, w2, b2):
    grid_spec = pltpu.PrefetchScalarGridSpec(
        num_scalar_prefetch=1,
        grid=(NT,),
        in_specs=[
            pl.BlockSpec((T, H), lambda i, te: (i, 0)),
            pl.BlockSpec((1, H, D), lambda i, te: (te[i], 0, 0)),
            pl.BlockSpec((1, 1, D), lambda i, te: (te[i], 0, 0)),
        ],
        out_specs=pl.BlockSpec((T, D), lambda i, te: (i, 0)),
    )
    return pl.pallas_call(
        _ffn2_body,
        grid_spec=grid_spec,
        out_shape=jax.ShapeDtypeStruct((NP, D), jnp.float32),
    )(te, h, w2, b2.reshape(E, 1, D))


def kernel(x, gate_w, gate_b, w1, b1, w2, b2):
    x_flat = x.reshape(S, D)
    e0, e1, g0, g1 = _gate(x_flat, gate_w, gate_b)
    disp_tok, pos0, pos1, te = _route_jax(e0, e1)
    xg = x_flat[disp_tok]
    h = _ffn1(te, xg, w1, b1)
    y = _ffn2(te, h, w2, b2)
    out = g0[:, None] * y[pos0] + g1[:, None] * y[pos1]
    return out.reshape(1, S, D)
